# Initial kernel scaffold; baseline (speedup 1.0000x reference)
import jax
import jax.numpy as jnp
from jax.experimental import pallas as pl


def kernel(x, edge_index, W1, b1, W2, b2, W3, b3):
    raise NotImplementedError("write your pallas kernel here")


# speedup vs baseline: 2.0965x; 2.0965x over previous
import functools

import jax
import jax.numpy as jnp
from jax.experimental import pallas as pl
from jax.experimental.pallas import tpu as pltpu

N = 10000
E = 160000
BLK = 2000


def _mm_body(x_ref, w_ref, o_ref):
    o_ref[...] = jnp.dot(x_ref[...], w_ref[...],
                         preferred_element_type=jnp.float32)


def _mm(x, w):
    m, k = x.shape
    n = w.shape[1]
    return pl.pallas_call(
        _mm_body,
        grid=(m // BLK,),
        in_specs=[pl.BlockSpec((BLK, k), lambda i: (i, 0)),
                  pl.BlockSpec((k, n), lambda i: (0, 0))],
        out_specs=pl.BlockSpec((BLK, n), lambda i: (i, 0)),
        out_shape=jax.ShapeDtypeStruct((m, n), jnp.float32),
    )(x, w)


def _layer(h, src, dst, dinv, W, b):
    t = _mm(h, W)
    g = dinv[:, None] * t
    agg = jax.ops.segment_sum(jnp.take(g, src, axis=0), dst, num_segments=N)
    z = dinv[:, None] * (agg + g) + b
    return jax.nn.leaky_relu(z, negative_slope=0.01)


def kernel(x, edge_index, W1, b1, W2, b2, W3, b3):
    src = edge_index[0]
    dst = edge_index[1]
    deg = jax.ops.segment_sum(jnp.ones((E,), jnp.float32), dst,
                              num_segments=N) + 1.0
    dinv = jax.lax.rsqrt(deg)
    h = _layer(x, src, dst, dinv, W1, b1)
    h = _layer(h, src, dst, dinv, W2, b2)
    h = _layer(h, src, dst, dinv, W3, b3)
    return h


# speedup vs baseline: 7.0083x; 3.3428x over previous
import functools

import jax
import jax.numpy as jnp
from jax import lax
from jax.experimental import pallas as pl
from jax.experimental.pallas import tpu as pltpu
from jax.experimental.pallas import tpu_sc as plsc

N = 10000
E = 160000
CH = 128
PADE = 163840
NACC = N + 16
BLK = 2000

_MESH = plsc.VectorSubcoreMesh(core_axis_name="c", subcore_axis_name="s")


def _deg_body(dst_hbm, zeros_hbm, ones_hbm, out_hbm,
              acc, ones_v, ida, idb, sa, sb):
    c = lax.axis_index("c")
    s = lax.axis_index("s")
    @pl.when(s < 10)
    def _():
        pltpu.sync_copy(zeros_hbm.at[pl.ds(s * 1024, 1024)],
                        acc.at[pl.ds(s * 1024, 1024)])
    pltpu.sync_copy(ones_hbm, ones_v)
    plsc.subcore_barrier()

    base = c * (PADE // 2) + s * (40 * CH)

    def _load(idx_ref, k):
        pltpu.sync_copy(dst_hbm.at[pl.ds(base + k * CH, CH)], idx_ref)

    def _sadd(idx_ref, sem):
        pltpu.async_copy(ones_v, acc.at[idx_ref], sem, add=True)

    def _swait(idx_ref, sem):
        pltpu.make_async_copy(ones_v, acc.at[idx_ref], sem).wait()

    _load(ida, 0)

    def body(k2, carry):
        a = 2 * k2

        @pl.when(k2 > 0)
        def _():
            _swait(idb, sb)
        _load(idb, a + 1)
        _sadd(ida, sa)
        _swait(ida, sa)

        @pl.when(k2 < 19)
        def _():
            _load(ida, a + 2)
        _sadd(idb, sb)
        return carry

    lax.fori_loop(0, 20, body, 0)
    _swait(idb, sb)
    plsc.subcore_barrier()

    @pl.when(s < 10)
    def _():
        pltpu.sync_copy(acc.at[pl.ds(s * 1024, 1024)],
                        out_hbm.at[pl.ds(c * 10240 + s * 1024, 1024)])


@functools.partial(
    pl.kernel,
    out_type=jax.ShapeDtypeStruct((20480,), jnp.float32),
    mesh=_MESH,
    scratch_types=[
        pltpu.VMEM_SHARED((10240,), jnp.float32),
        pltpu.VMEM((CH,), jnp.float32),
        pltpu.VMEM((CH,), jnp.int32),
        pltpu.VMEM((CH,), jnp.int32),
        pltpu.SemaphoreType.DMA,
        pltpu.SemaphoreType.DMA,
    ],
)
def _deg_sc(dst_hbm, zeros_hbm, ones_hbm, out_hbm, *scratch):
    _deg_body(dst_hbm, zeros_hbm, ones_hbm, out_hbm, *scratch)


def _edge_pipeline(gflat_hbm, srcflat_hbm, dst_hbm, acc, src_base, dst_base,
                   npairs, ia, ib, da, db, ra, rb, gsa, gsb, ssa, ssb):
    def _loadidx(isrc_ref, idst_ref, k):
        pltpu.sync_copy(srcflat_hbm.at[pl.ds(src_base + k * CH, CH)], isrc_ref)
        pltpu.sync_copy(dst_hbm.at[pl.ds(dst_base + k * CH, CH)], idst_ref)

    def _gstart(isrc_ref, rows_ref, sem):
        pltpu.async_copy(gflat_hbm.at[isrc_ref], rows_ref, sem)

    def _gwait(isrc_ref, rows_ref, sem):
        pltpu.make_async_copy(gflat_hbm.at[isrc_ref], rows_ref, sem).wait()

    def _sstart(rows_ref, idst_ref, sem):
        pltpu.async_copy(rows_ref, acc.at[idst_ref], sem, add=True)

    def _swait(rows_ref, idst_ref, sem):
        pltpu.make_async_copy(rows_ref, acc.at[idst_ref], sem).wait()

    _loadidx(ia, da, 0)
    _gstart(ia, ra, gsa)

    def body(k2, carry):
        a = 2 * k2

        @pl.when(k2 > 0)
        def _():
            _swait(rb, db, ssb)
        _loadidx(ib, db, a + 1)
        _gstart(ib, rb, gsb)
        _gwait(ia, ra, gsa)
        _sstart(ra, da, ssa)
        _gwait(ib, rb, gsb)
        _swait(ra, da, ssa)

        @pl.when(k2 < npairs - 1)
        def _():
            _loadidx(ia, da, a + 2)
            _gstart(ia, ra, gsa)
        _sstart(rb, db, ssb)
        return carry

    lax.fori_loop(0, npairs, body, 0)
    _swait(rb, db, ssb)


def _agg_col_body(gflat_hbm, srcflat_hbm, dst_hbm, out_hbm, acc, *bufs):
    c = lax.axis_index("c")
    s = lax.axis_index("s")

    @pl.when(s < 10)
    def _():
        pltpu.sync_copy(gflat_hbm.at[pl.ds(c * N + s * 1000, 1000)],
                        acc.at[pl.ds(s * 1000, 1000)])
    plsc.subcore_barrier()

    src_base = c * PADE + s * (80 * CH)
    dst_base = s * (80 * CH)
    _edge_pipeline(gflat_hbm, srcflat_hbm, dst_hbm, acc,
                   src_base, dst_base, 40, *bufs)
    plsc.subcore_barrier()

    @pl.when(s < 10)
    def _():
        pltpu.sync_copy(acc.at[pl.ds(s * 1000, 1000)],
                        out_hbm.at[c, pl.ds(s * 1000, 1000)])


def _agg_edge_body(gflat_hbm, srcflat_hbm, dst_hbm, out_hbm, acc, *bufs):
    c = lax.axis_index("c")
    s = lax.axis_index("s")

    @pl.when(s < 10)
    def _():
        pltpu.sync_copy(gflat_hbm.at[pl.ds(s * 1000, 1000)],
                        acc.at[pl.ds(s * 1000, 1000)])
    plsc.subcore_barrier()

    base = c * (PADE // 2) + s * (40 * CH)
    _edge_pipeline(gflat_hbm, srcflat_hbm, dst_hbm, acc,
                   base, base, 20, *bufs)
    plsc.subcore_barrier()

    @pl.when(s < 10)
    def _():
        pltpu.sync_copy(acc.at[pl.ds(s * 1000, 1000)],
                        out_hbm.at[c, pl.ds(s * 1000, 1000)])


def _agg_scratch(H):
    return [
        pltpu.VMEM_SHARED((NACC, H), jnp.float32),
        pltpu.VMEM((CH,), jnp.int32),
        pltpu.VMEM((CH,), jnp.int32),
        pltpu.VMEM((CH,), jnp.int32),
        pltpu.VMEM((CH,), jnp.int32),
        pltpu.VMEM((CH, H), jnp.float32),
        pltpu.VMEM((CH, H), jnp.float32),
        pltpu.SemaphoreType.DMA,
        pltpu.SemaphoreType.DMA,
        pltpu.SemaphoreType.DMA,
        pltpu.SemaphoreType.DMA,
    ]


_agg_col = functools.partial(
    pl.kernel,
    out_type=jax.ShapeDtypeStruct((2, N, 128), jnp.float32),
    mesh=_MESH,
    scratch_types=_agg_scratch(128),
)(_agg_col_body)

_agg_edge = functools.partial(
    pl.kernel,
    out_type=jax.ShapeDtypeStruct((2, N, 128), jnp.float32),
    mesh=_MESH,
    scratch_types=_agg_scratch(128),
)(_agg_edge_body)


def _leaky(z):
    return jnp.where(z >= 0, z, z * 0.01)


def _first_body(x_ref, w_ref, degp_ref, o_ref):
    dinv = lax.rsqrt(degp_ref[:, 0:1] + degp_ref[:, 1:2] + 1.0)
    t = jnp.dot(x_ref[...], w_ref[...], preferred_element_type=jnp.float32)
    g = dinv * t
    h = t.shape[1] // 2
    o_ref[0] = g[:, :h]
    o_ref[1] = g[:, h:]


def _first_tc(x, W, degp):
    d = W.shape[1]
    return pl.pallas_call(
        _first_body,
        grid=(N // BLK,),
        in_specs=[pl.BlockSpec((BLK, x.shape[1]), lambda i: (i, 0)),
                  pl.BlockSpec(W.shape, lambda i: (0, 0)),
                  pl.BlockSpec((BLK, 2), lambda i: (i, 0))],
        out_specs=pl.BlockSpec((2, BLK, d // 2), lambda i: (0, i, 0)),
        out_shape=jax.ShapeDtypeStruct((2, N, d // 2), jnp.float32),
    )(x, W, degp)


def _mid_body(split, sagg_ref, w_ref, degp_ref, b_ref, o_ref):
    H = sagg_ref.shape[2]
    dinv = lax.rsqrt(degp_ref[:, 0:1] + degp_ref[:, 1:2] + 1.0)
    hL = _leaky(dinv * sagg_ref[0] + b_ref[:, :H])
    hR = _leaky(dinv * sagg_ref[1] + b_ref[:, H:])
    t = (jnp.dot(hL, w_ref[:H, :], preferred_element_type=jnp.float32)
         + jnp.dot(hR, w_ref[H:, :], preferred_element_type=jnp.float32))
    g = dinv * t
    if split:
        h2 = t.shape[1] // 2
        o_ref[0] = g[:, :h2]
        o_ref[1] = g[:, h2:]
    else:
        o_ref[...] = g


def _mid_tc(sagg, W, degp, b, split):
    H = sagg.shape[2]
    d = W.shape[1]
    if split:
        out_specs = pl.BlockSpec((2, BLK, d // 2), lambda i: (0, i, 0))
        out_shape = jax.ShapeDtypeStruct((2, N, d // 2), jnp.float32)
    else:
        out_specs = pl.BlockSpec((BLK, d), lambda i: (i, 0))
        out_shape = jax.ShapeDtypeStruct((N, d), jnp.float32)
    return pl.pallas_call(
        functools.partial(_mid_body, split),
        grid=(N // BLK,),
        in_specs=[pl.BlockSpec((2, BLK, H), lambda i: (0, i, 0)),
                  pl.BlockSpec(W.shape, lambda i: (0, 0)),
                  pl.BlockSpec((BLK, 2), lambda i: (i, 0)),
                  pl.BlockSpec((1, 2 * H), lambda i: (0, 0))],
        out_specs=out_specs,
        out_shape=out_shape,
    )(sagg, W, degp, b)


def _last_body(p_ref, g_ref, degp_ref, b_ref, o_ref):
    dinv = lax.rsqrt(degp_ref[:, 0:1] + degp_ref[:, 1:2] + 1.0)
    sagg = p_ref[0] + p_ref[1] - g_ref[...]
    o_ref[...] = _leaky(dinv * sagg + b_ref[...])


def _last_tc(p, g, degp, b):
    d = g.shape[1]
    return pl.pallas_call(
        _last_body,
        grid=(N // BLK,),
        in_specs=[pl.BlockSpec((2, BLK, d), lambda i: (0, i, 0)),
                  pl.BlockSpec((BLK, d), lambda i: (i, 0)),
                  pl.BlockSpec((BLK, 2), lambda i: (i, 0)),
                  pl.BlockSpec((1, d), lambda i: (0, 0))],
        out_specs=pl.BlockSpec((BLK, d), lambda i: (i, 0)),
        out_shape=jax.ShapeDtypeStruct((N, d), jnp.float32),
    )(p, g, degp, b)


def kernel(x, edge_index, W1, b1, W2, b2, W3, b3):
    src = edge_index[0]
    dst = edge_index[1]
    pad = PADE - E
    srcp = jnp.concatenate([src, jnp.zeros((pad,), jnp.int32)])
    srcflat = jnp.concatenate([srcp, srcp + N])
    dstp = jnp.concatenate([dst, jnp.full((pad,), N, jnp.int32)])

    degp = _deg_sc(dstp, jnp.zeros((10240,), jnp.float32),
                   jnp.ones((CH,), jnp.float32))
    degp = degp.reshape(2, 10240)[:, :N].T

    g = _first_tc(x, W1, degp)
    sagg = _agg_col(g.reshape(2 * N, 128), srcflat, dstp)
    g = _mid_tc(sagg, W2, degp, b1.reshape(1, -1), True)
    sagg = _agg_col(g.reshape(2 * N, 128), srcflat, dstp)
    g = _mid_tc(sagg, W3, degp, b2.reshape(1, -1), False)
    p = _agg_edge(g, srcflat, dstp)
    return _last_tc(p, g, degp, b3.reshape(1, -1))


# speedup vs baseline: 7.8395x; 1.1186x over previous
import functools

import jax
import jax.numpy as jnp
from jax import lax
from jax.experimental import pallas as pl
from jax.experimental.pallas import tpu as pltpu
from jax.experimental.pallas import tpu_sc as plsc

N = 10000
E = 160000
CH = 128
PADE = 163840
NACC = N + 16
BLK = 2000

_MESH = plsc.VectorSubcoreMesh(core_axis_name="c", subcore_axis_name="s")


def _deg_body(dst_hbm, zeros_hbm, ones_hbm, out_hbm,
              acc, ones_v, dbuf, sa, sb):
    c = lax.axis_index("c")
    s = lax.axis_index("s")
    @pl.when(s < 10)
    def _():
        pltpu.sync_copy(zeros_hbm.at[pl.ds(s * 1024, 1024)],
                        acc.at[pl.ds(s * 1024, 1024)])
    pltpu.sync_copy(ones_hbm, ones_v)
    pltpu.sync_copy(dst_hbm.at[pl.ds(c * 640 + s * 40, 40)], dbuf)
    plsc.subcore_barrier()

    def _sadd(j, sem):
        pltpu.async_copy(ones_v, acc.at[dbuf.at[j]], sem, add=True)

    def _swait(j, sem):
        pltpu.make_async_copy(ones_v, acc.at[dbuf.at[j]], sem).wait()

    _sadd(0, sa)

    def body(k2, carry):
        a = 2 * k2

        @pl.when(k2 > 0)
        def _():
            _swait(a - 1, sb)
        _sadd(a + 1, sb)
        _swait(a, sa)

        @pl.when(k2 < 19)
        def _():
            _sadd(a + 2, sa)
        return carry

    lax.fori_loop(0, 20, body, 0)
    _swait(39, sb)
    plsc.subcore_barrier()

    @pl.when(s < 10)
    def _():
        pltpu.sync_copy(acc.at[pl.ds(s * 1024, 1024)],
                        out_hbm.at[pl.ds(c * 10240 + s * 1024, 1024)])


@functools.partial(
    pl.kernel,
    out_type=jax.ShapeDtypeStruct((20480,), jnp.float32),
    mesh=_MESH,
    scratch_types=[
        pltpu.VMEM_SHARED((10240,), jnp.float32),
        pltpu.VMEM((CH,), jnp.float32),
        pltpu.VMEM((40, CH), jnp.int32),
        pltpu.SemaphoreType.DMA,
        pltpu.SemaphoreType.DMA,
    ],
)
def _deg_sc(dst_hbm, zeros_hbm, ones_hbm, out_hbm, *scratch):
    _deg_body(dst_hbm, zeros_hbm, ones_hbm, out_hbm, *scratch)


def _edge_pipeline(gflat_hbm, acc, sbuf, dbuf, nchunks,
                   ra, rb, gsa, gsb, ssa, ssb):
    def _gstart(rows_ref, sem, j):
        pltpu.async_copy(gflat_hbm.at[sbuf.at[j]], rows_ref, sem)

    def _gwait(rows_ref, sem, j):
        pltpu.make_async_copy(gflat_hbm.at[sbuf.at[j]], rows_ref, sem).wait()

    def _sstart(rows_ref, sem, j):
        pltpu.async_copy(rows_ref, acc.at[dbuf.at[j]], sem, add=True)

    def _swait(rows_ref, sem, j):
        pltpu.make_async_copy(rows_ref, acc.at[dbuf.at[j]], sem).wait()

    _gstart(ra, gsa, 0)

    def body(k2, carry):
        a = 2 * k2

        @pl.when(k2 > 0)
        def _():
            _swait(rb, ssb, a - 1)
        _gstart(rb, gsb, a + 1)
        _gwait(ra, gsa, a)
        _sstart(ra, ssa, a)
        _gwait(rb, gsb, a + 1)
        _swait(ra, ssa, a)

        @pl.when(k2 < nchunks // 2 - 1)
        def _():
            _gstart(ra, gsa, a + 2)
        _sstart(rb, ssb, a + 1)
        return carry

    lax.fori_loop(0, nchunks // 2, body, 0)
    _swait(rb, ssb, nchunks - 1)


def _agg_col_body(gflat_hbm, srcflat_hbm, dst_hbm, out_hbm,
                  acc, sbuf, dbuf, *bufs):
    c = lax.axis_index("c")
    s = lax.axis_index("s")

    @pl.when(s < 10)
    def _():
        pltpu.sync_copy(gflat_hbm.at[pl.ds(c * N + s * 1000, 1000)],
                        acc.at[pl.ds(s * 1000, 1000)])
    plsc.subcore_barrier()

    for h in range(2):
        pltpu.sync_copy(
            srcflat_hbm.at[pl.ds(c * 1280 + s * 80 + h * 40, 40)], sbuf)
        pltpu.sync_copy(dst_hbm.at[pl.ds(s * 80 + h * 40, 40)], dbuf)
        _edge_pipeline(gflat_hbm, acc, sbuf, dbuf, 40, *bufs)
    plsc.subcore_barrier()

    @pl.when(s < 10)
    def _():
        pltpu.sync_copy(acc.at[pl.ds(s * 1000, 1000)],
                        out_hbm.at[c, pl.ds(s * 1000, 1000)])


def _agg_edge_body(gflat_hbm, srcflat_hbm, dst_hbm, out_hbm,
                   acc, sbuf, dbuf, *bufs):
    c = lax.axis_index("c")
    s = lax.axis_index("s")

    @pl.when(s < 10)
    def _():
        pltpu.sync_copy(gflat_hbm.at[pl.ds(s * 1000, 1000)],
                        acc.at[pl.ds(s * 1000, 1000)])
    base = c * 640 + s * 40
    pltpu.sync_copy(srcflat_hbm.at[pl.ds(base, 40)], sbuf)
    pltpu.sync_copy(dst_hbm.at[pl.ds(base, 40)], dbuf)
    plsc.subcore_barrier()

    _edge_pipeline(gflat_hbm, acc, sbuf, dbuf, 40, *bufs)
    plsc.subcore_barrier()

    @pl.when(s < 10)
    def _():
        pltpu.sync_copy(acc.at[pl.ds(s * 1000, 1000)],
                        out_hbm.at[c, pl.ds(s * 1000, 1000)])


def _agg_scratch(H):
    return [
        pltpu.VMEM_SHARED((NACC, H), jnp.float32),
        pltpu.VMEM((40, CH), jnp.int32),
        pltpu.VMEM((40, CH), jnp.int32),
        pltpu.VMEM((CH, H), jnp.float32),
        pltpu.VMEM((CH, H), jnp.float32),
        pltpu.SemaphoreType.DMA,
        pltpu.SemaphoreType.DMA,
        pltpu.SemaphoreType.DMA,
        pltpu.SemaphoreType.DMA,
    ]


_agg_col = functools.partial(
    pl.kernel,
    out_type=jax.ShapeDtypeStruct((2, N, 128), jnp.float32),
    mesh=_MESH,
    scratch_types=_agg_scratch(128),
)(_agg_col_body)

_agg_edge = functools.partial(
    pl.kernel,
    out_type=jax.ShapeDtypeStruct((2, N, 128), jnp.float32),
    mesh=_MESH,
    scratch_types=_agg_scratch(128),
)(_agg_edge_body)


def _leaky(z):
    return jnp.where(z >= 0, z, z * 0.01)


def _first_body(x_ref, w_ref, degp_ref, o_ref):
    dinv = lax.rsqrt(degp_ref[:, 0:1] + degp_ref[:, 1:2] + 1.0)
    t = jnp.dot(x_ref[...], w_ref[...], preferred_element_type=jnp.float32)
    g = dinv * t
    h = t.shape[1] // 2
    o_ref[0] = g[:, :h]
    o_ref[1] = g[:, h:]


def _first_tc(x, W, degp):
    d = W.shape[1]
    return pl.pallas_call(
        _first_body,
        grid=(N // BLK,),
        in_specs=[pl.BlockSpec((BLK, x.shape[1]), lambda i: (i, 0)),
                  pl.BlockSpec(W.shape, lambda i: (0, 0)),
                  pl.BlockSpec((BLK, 2), lambda i: (i, 0))],
        out_specs=pl.BlockSpec((2, BLK, d // 2), lambda i: (0, i, 0)),
        out_shape=jax.ShapeDtypeStruct((2, N, d // 2), jnp.float32),
    )(x, W, degp)


def _mid_body(split, sagg_ref, w_ref, degp_ref, b_ref, o_ref):
    H = sagg_ref.shape[2]
    dinv = lax.rsqrt(degp_ref[:, 0:1] + degp_ref[:, 1:2] + 1.0)
    hL = _leaky(dinv * sagg_ref[0] + b_ref[:, :H])
    hR = _leaky(dinv * sagg_ref[1] + b_ref[:, H:])
    t = (jnp.dot(hL, w_ref[:H, :], preferred_element_type=jnp.float32)
         + jnp.dot(hR, w_ref[H:, :], preferred_element_type=jnp.float32))
    g = dinv * t
    if split:
        h2 = t.shape[1] // 2
        o_ref[0] = g[:, :h2]
        o_ref[1] = g[:, h2:]
    else:
        o_ref[...] = g


def _mid_tc(sagg, W, degp, b, split):
    H = sagg.shape[2]
    d = W.shape[1]
    if split:
        out_specs = pl.BlockSpec((2, BLK, d // 2), lambda i: (0, i, 0))
        out_shape = jax.ShapeDtypeStruct((2, N, d // 2), jnp.float32)
    else:
        out_specs = pl.BlockSpec((BLK, d), lambda i: (i, 0))
        out_shape = jax.ShapeDtypeStruct((N, d), jnp.float32)
    return pl.pallas_call(
        functools.partial(_mid_body, split),
        grid=(N // BLK,),
        in_specs=[pl.BlockSpec((2, BLK, H), lambda i: (0, i, 0)),
                  pl.BlockSpec(W.shape, lambda i: (0, 0)),
                  pl.BlockSpec((BLK, 2), lambda i: (i, 0)),
                  pl.BlockSpec((1, 2 * H), lambda i: (0, 0))],
        out_specs=out_specs,
        out_shape=out_shape,
    )(sagg, W, degp, b)


def _last_body(p_ref, g_ref, degp_ref, b_ref, o_ref):
    dinv = lax.rsqrt(degp_ref[:, 0:1] + degp_ref[:, 1:2] + 1.0)
    sagg = p_ref[0] + p_ref[1] - g_ref[...]
    o_ref[...] = _leaky(dinv * sagg + b_ref[...])


def _last_tc(p, g, degp, b):
    d = g.shape[1]
    return pl.pallas_call(
        _last_body,
        grid=(N // BLK,),
        in_specs=[pl.BlockSpec((2, BLK, d), lambda i: (0, i, 0)),
                  pl.BlockSpec((BLK, d), lambda i: (i, 0)),
                  pl.BlockSpec((BLK, 2), lambda i: (i, 0)),
                  pl.BlockSpec((1, d), lambda i: (0, 0))],
        out_specs=pl.BlockSpec((BLK, d), lambda i: (i, 0)),
        out_shape=jax.ShapeDtypeStruct((N, d), jnp.float32),
    )(p, g, degp, b)


def kernel(x, edge_index, W1, b1, W2, b2, W3, b3):
    src = edge_index[0]
    dst = edge_index[1]
    pad = PADE - E
    srcp = jnp.concatenate([src, jnp.zeros((pad,), jnp.int32)])
    srcflat = jnp.concatenate([srcp, srcp + N]).reshape(2560, CH)
    dstp = jnp.concatenate(
        [dst, jnp.full((pad,), N, jnp.int32)]).reshape(1280, CH)

    degp = _deg_sc(dstp, jnp.zeros((10240,), jnp.float32),
                   jnp.ones((CH,), jnp.float32))
    degp = degp.reshape(2, 10240)[:, :N].T

    g = _first_tc(x, W1, degp)
    sagg = _agg_col(g.reshape(2 * N, 128), srcflat, dstp)
    g = _mid_tc(sagg, W2, degp, b1.reshape(1, -1), True)
    sagg = _agg_col(g.reshape(2 * N, 128), srcflat, dstp)
    g = _mid_tc(sagg, W3, degp, b2.reshape(1, -1), False)
    p = _agg_edge(g, srcflat, dstp)
    return _last_tc(p, g, degp, b3.reshape(1, -1))
